# Initial kernel scaffold; baseline (speedup 1.0000x reference)
import jax
import jax.numpy as jnp
from jax.experimental import pallas as pl


def kernel(x, durations):
    raise NotImplementedError("write your pallas kernel here")


# speedup vs baseline: 16.6202x; 16.6202x over previous
import functools

import jax
import jax.numpy as jnp
from jax import lax
from jax.experimental import pallas as pl
from jax.experimental.pallas import tpu as pltpu
from jax.experimental.pallas import tpu_sc as plsc

NB = 16
NF = 80
NT = 8192
NS = 512
MAXD = 15
ROWS_PER_W = 40

_mesh = plsc.VectorSubcoreMesh(core_axis_name="c", subcore_axis_name="s")


@functools.partial(
    pl.kernel,
    mesh=_mesh,
    compiler_params=pltpu.CompilerParams(needs_layout_passes=False),
    out_type=jax.ShapeDtypeStruct((NB, NF, NS), jnp.float32),
    scratch_types=[
        pltpu.VMEM((NS,), jnp.int32),
        pltpu.VMEM((NS,), jnp.int32),
        pltpu.VMEM((NT,), jnp.float32),
        pltpu.VMEM((ROWS_PER_W, NS), jnp.float32),
    ],
)
def _seg_avg(x_hbm, dur_hbm, out_hbm, dur_v, starts_v, row_v, out_v):
    cid = lax.axis_index("c")
    sid = lax.axis_index("s")
    wid = sid * 2 + cid
    batch = wid // 2
    f0 = (wid % 2) * ROWS_PER_W

    pltpu.sync_copy(dur_hbm.at[batch], dur_v)

    iota = jnp.arange(16, dtype=jnp.int32)
    lane15 = jnp.full((16,), 15, jnp.int32)

    def bounds_body(g, carry_v):
        base = pl.multiple_of(g * 16, 16)
        d = dur_v[pl.ds(base, 16)]
        ends = d
        for k in (1, 2, 4, 8):
            sh = ends.at[jnp.maximum(iota - k, 0)].get(mode="promise_in_bounds")
            ends = ends + jnp.where(iota >= k, sh, 0)
        ends = ends + carry_v
        starts_v[pl.ds(base, 16)] = ends - d
        return ends.at[lane15].get(mode="promise_in_bounds")

    lax.fori_loop(0, NS // 16, bounds_body, jnp.zeros((16,), jnp.int32))

    def row_body(r, _):
        pltpu.sync_copy(x_hbm.at[batch, f0 + r], row_v)

        def grp_body(g, _2):
            base = pl.multiple_of(g * 16, 16)
            starts = starts_v[pl.ds(base, 16)]
            lens = dur_v[pl.ds(base, 16)]
            acc = jnp.zeros((16,), jnp.float32)
            cnt = jnp.zeros((16,), jnp.float32)
            for d in range(MAXD):
                m = lens > d
                v = plsc.load_gather(row_v, [starts + d])
                keep = m & (v != 0.0)
                acc = acc + jnp.where(keep, v, 0.0)
                cnt = cnt + jnp.where(keep, 1.0, 0.0)
            avg = jnp.where(cnt > 0.0, acc / jnp.where(cnt > 0.0, cnt, 1.0), 0.0)
            out_v[r, pl.ds(base, 16)] = avg
            return 0

        lax.fori_loop(0, NS // 16, grp_body, 0)
        return 0

    lax.fori_loop(0, ROWS_PER_W, row_body, 0)
    pltpu.sync_copy(out_v, out_hbm.at[batch, pl.ds(f0, ROWS_PER_W)])


def kernel(x, durations):
    return _seg_avg(x, durations.astype(jnp.int32))


# speedup vs baseline: 23.3697x; 1.4061x over previous
import functools

import jax
import jax.numpy as jnp
from jax import lax
from jax.experimental import pallas as pl
from jax.experimental.pallas import tpu as pltpu
from jax.experimental.pallas import tpu_sc as plsc

NB = 16
NF = 80
NT = 8192
NS = 512
MAXD = 15
ROWS_PER_W = 40

_mesh = plsc.VectorSubcoreMesh(core_axis_name="c", subcore_axis_name="s")


@functools.partial(
    pl.kernel,
    mesh=_mesh,
    compiler_params=pltpu.CompilerParams(needs_layout_passes=False),
    out_type=jax.ShapeDtypeStruct((NB, NF, NS), jnp.float32),
    scratch_types=[
        pltpu.VMEM((NS,), jnp.int32),
        pltpu.VMEM((NS,), jnp.int32),
        pltpu.VMEM((NT,), jnp.float32),
        pltpu.VMEM((NT,), jnp.float32),
        pltpu.VMEM((ROWS_PER_W, NS), jnp.float32),
        pltpu.SemaphoreType.DMA,
        pltpu.SemaphoreType.DMA,
    ],
)
def _seg_avg(x_hbm, dur_hbm, out_hbm, dur_v, starts_v, row_a, row_b, out_v,
             sem_a, sem_b):
    cid = lax.axis_index("c")
    sid = lax.axis_index("s")
    wid = sid * 2 + cid
    batch = wid // 2
    f0 = (wid % 2) * ROWS_PER_W

    pltpu.sync_copy(dur_hbm.at[batch], dur_v)

    iota = jnp.arange(16, dtype=jnp.int32)
    lane15 = jnp.full((16,), 15, jnp.int32)

    def bounds_body(g, carry_v):
        base = pl.multiple_of(g * 16, 16)
        d = dur_v[pl.ds(base, 16)]
        ends = d
        for k in (1, 2, 4, 8):
            sh = ends.at[jnp.maximum(iota - k, 0)].get(mode="promise_in_bounds")
            ends = ends + jnp.where(iota >= k, sh, 0)
        ends = ends + carry_v
        starts_v[pl.ds(base, 16)] = ends - d
        return ends.at[lane15].get(mode="promise_in_bounds")

    lax.fori_loop(0, NS // 16, bounds_body, jnp.zeros((16,), jnp.int32))

    def compute_row(row_v, r):
        def grp_body(g, _2):
            base = pl.multiple_of(g * 16, 16)
            starts = starts_v[pl.ds(base, 16)]
            lens = dur_v[pl.ds(base, 16)]
            acc = jnp.zeros((16,), jnp.float32)
            cnt = jnp.zeros((16,), jnp.float32)
            for d in range(MAXD):
                m = lens > d
                v = plsc.load_gather(row_v, [starts + d])
                keep = m & (v != 0.0)
                acc = acc + jnp.where(keep, v, 0.0)
                cnt = cnt + jnp.where(keep, 1.0, 0.0)
            avg = jnp.where(cnt > 0.0, acc / jnp.where(cnt > 0.0, cnt, 1.0), 0.0)
            out_v[r, pl.ds(base, 16)] = avg
            return 0

        lax.fori_loop(0, NS // 16, grp_body, 0)

    pltpu.async_copy(x_hbm.at[batch, f0], row_a, sem_a)

    def pair_body(p, _):
        r0 = 2 * p
        pltpu.async_copy(x_hbm.at[batch, f0 + r0 + 1], row_b, sem_b)
        pltpu.make_async_copy(x_hbm.at[batch, f0 + r0], row_a, sem_a).wait()
        compute_row(row_a, r0)

        @pl.when(p < ROWS_PER_W // 2 - 1)
        def _prefetch():
            pltpu.async_copy(x_hbm.at[batch, f0 + r0 + 2], row_a, sem_a)

        pltpu.make_async_copy(x_hbm.at[batch, f0 + r0 + 1], row_b, sem_b).wait()
        compute_row(row_b, r0 + 1)
        return 0

    lax.fori_loop(0, ROWS_PER_W // 2, pair_body, 0)
    pltpu.sync_copy(out_v, out_hbm.at[batch, pl.ds(f0, ROWS_PER_W)])


def kernel(x, durations):
    return _seg_avg(x, durations.astype(jnp.int32))


# speedup vs baseline: 30.3039x; 1.2967x over previous
import functools

import jax
import jax.numpy as jnp
from jax import lax
from jax.experimental import pallas as pl
from jax.experimental.pallas import tpu as pltpu
from jax.experimental.pallas import tpu_sc as plsc

NB = 16
NF = 80
NT = 8192
NS = 512
MAXD = 15
ROWS_PER_W = 40

_mesh = plsc.VectorSubcoreMesh(core_axis_name="c", subcore_axis_name="s")


@functools.partial(
    pl.kernel,
    mesh=_mesh,
    compiler_params=pltpu.CompilerParams(needs_layout_passes=False),
    out_type=jax.ShapeDtypeStruct((NB, NF, NS), jnp.float32),
    scratch_types=[
        pltpu.VMEM((NS,), jnp.int32),
        pltpu.VMEM((NS,), jnp.int32),
        pltpu.VMEM((NT + 16,), jnp.float32),
        pltpu.VMEM((NT + 16,), jnp.float32),
        pltpu.VMEM((ROWS_PER_W, NS), jnp.float32),
        pltpu.SemaphoreType.DMA,
        pltpu.SemaphoreType.DMA,
    ],
)
def _seg_avg(x_hbm, dur_hbm, out_hbm, dur_v, starts_v, row_a, row_b, out_v,
             sem_a, sem_b):
    cid = lax.axis_index("c")
    sid = lax.axis_index("s")
    wid = sid * 2 + cid
    batch = wid // 2
    f0 = (wid % 2) * ROWS_PER_W

    pltpu.sync_copy(dur_hbm.at[batch], dur_v)

    iota = jnp.arange(16, dtype=jnp.int32)
    lane15 = jnp.full((16,), 15, jnp.int32)

    def bounds_body(g, carry_v):
        base = pl.multiple_of(g * 16, 16)
        d = dur_v[pl.ds(base, 16)]
        ends = d
        for k in (1, 2, 4, 8):
            sh = ends.at[jnp.maximum(iota - k, 0)].get(mode="promise_in_bounds")
            ends = ends + jnp.where(iota >= k, sh, 0)
        ends = ends + carry_v
        starts_v[pl.ds(base, 16)] = ends - d
        return ends.at[lane15].get(mode="promise_in_bounds")

    lax.fori_loop(0, NS // 16, bounds_body, jnp.zeros((16,), jnp.int32))

    zeros = jnp.zeros((16,), jnp.float32)

    def compute_row(row_v, r):
        def grp_body(g, _2):
            base = pl.multiple_of(g * 32, 16)
            s0 = starts_v[pl.ds(base, 16)]
            l0 = dur_v[pl.ds(base, 16)]
            s1 = starts_v[pl.ds(base + 16, 16)]
            l1 = dur_v[pl.ds(base + 16, 16)]
            acc0 = cnt0 = acc1 = cnt1 = zeros
            for d in range(MAXD):
                i0 = jnp.where(l0 > d, s0 + d, NT)
                i1 = jnp.where(l1 > d, s1 + d, NT)
                v0 = plsc.load_gather(row_v, [i0])
                v1 = plsc.load_gather(row_v, [i1])
                acc0 = acc0 + v0
                acc1 = acc1 + v1
                cnt0 = cnt0 + jnp.where(v0 == 0.0, 0.0, 1.0)
                cnt1 = cnt1 + jnp.where(v1 == 0.0, 0.0, 1.0)
            out_v[r, pl.ds(base, 16)] = acc0 / jnp.maximum(cnt0, 1.0)
            out_v[r, pl.ds(base + 16, 16)] = acc1 / jnp.maximum(cnt1, 1.0)
            return 0

        lax.fori_loop(0, NS // 32, grp_body, 0)

    row_a[pl.ds(NT, 16)] = zeros
    row_b[pl.ds(NT, 16)] = zeros
    pltpu.async_copy(x_hbm.at[batch, f0], row_a.at[pl.ds(0, NT)], sem_a)

    def pair_body(p, _):
        r0 = 2 * p
        pltpu.async_copy(x_hbm.at[batch, f0 + r0 + 1], row_b.at[pl.ds(0, NT)], sem_b)
        pltpu.make_async_copy(x_hbm.at[batch, f0 + r0], row_a.at[pl.ds(0, NT)], sem_a).wait()
        compute_row(row_a, r0)

        @pl.when(p < ROWS_PER_W // 2 - 1)
        def _prefetch():
            pltpu.async_copy(x_hbm.at[batch, f0 + r0 + 2], row_a.at[pl.ds(0, NT)], sem_a)

        pltpu.make_async_copy(x_hbm.at[batch, f0 + r0 + 1], row_b.at[pl.ds(0, NT)], sem_b).wait()
        compute_row(row_b, r0 + 1)
        return 0

    lax.fori_loop(0, ROWS_PER_W // 2, pair_body, 0)
    pltpu.sync_copy(out_v, out_hbm.at[batch, pl.ds(f0, ROWS_PER_W)])


def kernel(x, durations):
    return _seg_avg(x, durations.astype(jnp.int32))
